# baseline (speedup 1.0000x reference)
import functools

import jax
import jax.numpy as jnp
from jax import lax
from jax.experimental import pallas as pl
from jax.experimental.pallas import tpu as pltpu
from jax.experimental.pallas import tpu_sc as plsc

T, B, D = 20, 65536, 16
B_EX = 16384
NC, NS = 2, 16
NW = NC * NS
CB = B // NW
K = 128
NK = CB // K
EROWS = B_EX // NS // K
VPR = K // 16
PAD = B


def _body1(ll, rll, fm, rm, fv, rv, eix, mll, mm, mv,
           eidx2, ones_v, zeros_v, mvalc, dst2, llc, rllc, bufF, bufR,
           mask_spm, semF, semR, semW, semS):
    cid = lax.axis_index("c")
    sid = lax.axis_index("s")
    wid = sid * NC + cid
    base = wid * CB
    iota16 = lax.broadcasted_iota(jnp.int32, (16,), 0)

    def fill(ref, nvec, val):
        def f(i, _):
            ref[pl.ds(i * 16, 16)] = jnp.full((16,), val, jnp.int32)
            return 0
        lax.fori_loop(0, nvec, f, 0)

    fill(ones_v, K // 16, 1)
    fill(zeros_v, (B // NS) // 16, 0)

    pltpu.sync_copy(zeros_v, mask_spm.at[pl.ds(sid * (B // NS), B // NS)])
    plsc.subcore_barrier()
    pltpu.sync_copy(eix.at[pl.ds(sid * EROWS, EROWS)], eidx2)
    for j in range(EROWS):
        pltpu.sync_copy(ones_v, mask_spm.at[eidx2.at[j]], add=True)
    plsc.subcore_barrier()

    pltpu.sync_copy(mask_spm.at[pl.ds(base, CB)], mvalc)

    def cbody(k, _):
        row = k // VPR
        off = (k % VPR) * 16
        mvec = mvalc[pl.ds(k * 16, 16)]
        jloc = k * 16 + iota16
        dump = B + base + jloc
        dst2[row, pl.ds(off, 16)] = jnp.where(mvec > 0, base + jloc, dump)
        return 0
    lax.fori_loop(0, CB // 16, cbody, 0)

    pltpu.sync_copy(ll.at[pl.ds(base, CB)], llc)
    pltpu.sync_copy(rll.at[pl.ds(base, CB)], rllc)

    def selb(k, _):
        mvec = mvalc[pl.ds(k * 16, 16)]
        a = llc[pl.ds(k * 16, 16)]
        b = rllc[pl.ds(k * 16, 16)]
        llc[pl.ds(k * 16, 16)] = jnp.where(mvec > 0, b, a)
        return 0
    lax.fori_loop(0, CB // 16, selb, 0)
    pltpu.sync_copy(llc, mll.at[pl.ds(base, CB)])

    def tbody(t, _):
        for (ftab, rtab, mtab) in ((fm, rm, mm), (fv, rv, mv)):
            for j in range(NK):
                pltpu.async_copy(ftab.at[t, pl.ds(base + j * K, K)],
                                 bufF.at[pl.ds(j * K, K)], semF)
                pltpu.async_copy(rtab.at[t, pl.ds(base + j * K, K)],
                                 bufR.at[pl.ds(j * K, K)], semR)
            for j in range(NK):
                pltpu.make_async_copy(ftab.at[t, pl.ds(base + j * K, K)],
                                      bufF.at[pl.ds(j * K, K)], semF).wait()
            for j in range(NK):
                pltpu.async_copy(bufF.at[pl.ds(j * K, K)],
                                 mtab.at[t, pl.ds(base + j * K, K)], semW)
            for j in range(NK):
                pltpu.make_async_copy(bufF.at[pl.ds(j * K, K)],
                                      mtab.at[t, pl.ds(base + j * K, K)],
                                      semW).wait()
            for j in range(NK):
                pltpu.make_async_copy(rtab.at[t, pl.ds(base + j * K, K)],
                                      bufR.at[pl.ds(j * K, K)], semR).wait()
            for j in range(NK):
                pltpu.async_copy(bufR.at[pl.ds(j * K, K)],
                                 mtab.at[t].at[dst2.at[j]], semS)
            for j in range(NK):
                pltpu.make_async_copy(bufR.at[pl.ds(j * K, K)],
                                      mtab.at[t].at[dst2.at[j]], semS).wait()
        return 0

    lax.fori_loop(0, T, tbody, 0)


def _body2(mll, mm, mv, rix, llo, om, ov, r2d, buf, llv2, semF, semG):
    cid = lax.axis_index("c")
    sid = lax.axis_index("s")
    wid = sid * NC + cid
    base = wid * CB

    pltpu.sync_copy(rix.at[pl.ds(wid * NK, NK)], r2d)

    for j in range(NK):
        pltpu.async_copy(mll.at[r2d.at[j]], llv2.at[j], semF)
    for j in range(NK):
        pltpu.make_async_copy(mll.at[r2d.at[j]], llv2.at[j], semF).wait()
    pltpu.sync_copy(llv2, llo.at[pl.ds(wid * NK, NK)])

    def tbody(t, _):
        for (mtab, otab) in ((mm, om), (mv, ov)):
            for j in range(NK):
                pltpu.async_copy(mtab.at[t].at[r2d.at[j]],
                                 buf.at[pl.ds(j * K, K)], semG)
            for j in range(NK):
                pltpu.make_async_copy(mtab.at[t].at[r2d.at[j]],
                                      buf.at[pl.ds(j * K, K)], semG).wait()
            pltpu.sync_copy(buf, otab.at[t, pl.ds(base, CB)])
        return 0

    lax.fori_loop(0, T, tbody, 0)


_mesh = plsc.VectorSubcoreMesh(core_axis_name="c", subcore_axis_name="s")
_params = pltpu.CompilerParams(use_tc_tiling_on_sc=False)

_k1 = functools.partial(
    pl.kernel,
    out_type=[
        jax.ShapeDtypeStruct((B,), jnp.float32),
        jax.ShapeDtypeStruct((T, B + PAD, D), jnp.float32),
        jax.ShapeDtypeStruct((T, B + PAD, D), jnp.float32),
    ],
    mesh=_mesh,
    compiler_params=_params,
    scratch_types=[
        pltpu.VMEM((EROWS, K), jnp.int32),
        pltpu.VMEM((K,), jnp.int32),
        pltpu.VMEM((B // NS,), jnp.int32),
        pltpu.VMEM((CB,), jnp.int32),
        pltpu.VMEM((NK, K), jnp.int32),
        pltpu.VMEM((CB,), jnp.float32),
        pltpu.VMEM((CB,), jnp.float32),
        pltpu.VMEM((CB, D), jnp.float32),
        pltpu.VMEM((CB, D), jnp.float32),
        pltpu.VMEM_SHARED((B,), jnp.int32),
        pltpu.SemaphoreType.DMA,
        pltpu.SemaphoreType.DMA,
        pltpu.SemaphoreType.DMA,
        pltpu.SemaphoreType.DMA,
    ],
)(_body1)

_k2 = functools.partial(
    pl.kernel,
    out_type=[
        jax.ShapeDtypeStruct((B // K, K), jnp.float32),
        jax.ShapeDtypeStruct((T, B, D), jnp.float32),
        jax.ShapeDtypeStruct((T, B, D), jnp.float32),
    ],
    mesh=_mesh,
    compiler_params=_params,
    scratch_types=[
        pltpu.VMEM((NK, K), jnp.int32),
        pltpu.VMEM((CB, D), jnp.float32),
        pltpu.VMEM((NK, K), jnp.float32),
        pltpu.SemaphoreType.DMA,
        pltpu.SemaphoreType.DMA,
    ],
)(_body2)


def kernel(loglikelihood, filter_means, filter_vars, res_loglikelihood,
           res_means, res_vars, exchange_indices, resample_indices):
    eix = exchange_indices.astype(jnp.int32).reshape(B_EX // K, K)
    rix = resample_indices.astype(jnp.int32).reshape(B // K, K)
    mll, mm, mv = _k1(loglikelihood, res_loglikelihood, filter_means,
                      res_means, filter_vars, res_vars, eix)
    llo, om, ov = _k2(mll, mm, mv, rix)
    return (llo.reshape(B), om, ov)


# speedup vs baseline: 1.0542x; 1.0542x over previous
import functools

import jax
import jax.numpy as jnp
from jax import lax
from jax.experimental import pallas as pl
from jax.experimental.pallas import tpu as pltpu
from jax.experimental.pallas import tpu_sc as plsc

T, B, D = 20, 65536, 16
B_EX = 16384
NC, NS = 2, 16
NW = NC * NS
CB = B // NW
K = 128
NK = CB // K
EROWS = B_EX // NS // K
VPR = K // 16


def _body(ll, fm, fv, rll, rm, rv, eix, rix, llo, om, ov,
          r2d, mval, eidx2, ones_v, zeros_v, dst2, bufFR,
          llv2, rllv2, mask_spm, spmem_out, semF, semR, semS):
    cid = lax.axis_index("c")
    sid = lax.axis_index("s")
    wid = sid * NC + cid
    base = wid * CB
    iota16 = lax.broadcasted_iota(jnp.int32, (16,), 0)

    def fill(ref, nvec, val):
        def f(i, _):
            ref[pl.ds(i * 16, 16)] = jnp.full((16,), val, jnp.int32)
            return 0
        lax.fori_loop(0, nvec, f, 0)

    fill(ones_v, K // 16, 1)
    fill(zeros_v, (B // NS) // 16, 0)

    pltpu.sync_copy(rix.at[pl.ds(wid * NK, NK)], r2d)

    pltpu.sync_copy(zeros_v, mask_spm.at[pl.ds(sid * (B // NS), B // NS)])
    plsc.subcore_barrier()
    pltpu.sync_copy(eix.at[pl.ds(sid * EROWS, EROWS)], eidx2)
    for j in range(EROWS):
        pltpu.sync_copy(ones_v, mask_spm.at[eidx2.at[j]], add=True)
    plsc.subcore_barrier()

    for j in range(NK):
        pltpu.async_copy(mask_spm.at[r2d.at[j]], mval.at[j], semF)
    for j in range(NK):
        pltpu.make_async_copy(mask_spm.at[r2d.at[j]], mval.at[j], semF).wait()

    slot = sid * (CB + K)
    def cbody(k, _):
        row = k // VPR
        off = (k % VPR) * 16
        mvec = mval[row, pl.ds(off, 16)]
        jloc = k * 16 + iota16
        dump = slot + CB + jnp.bitwise_and(jloc, K - 1)
        dst2[row, pl.ds(off, 16)] = jnp.where(mvec > 0, slot + jloc, dump)
        return 0
    lax.fori_loop(0, CB // 16, cbody, 0)

    for j in range(NK):
        pltpu.async_copy(ll.at[r2d.at[j]], llv2.at[j], semF)
        pltpu.async_copy(rll.at[r2d.at[j]], rllv2.at[j], semR)
    for j in range(NK):
        pltpu.make_async_copy(ll.at[r2d.at[j]], llv2.at[j], semF).wait()
        pltpu.make_async_copy(rll.at[r2d.at[j]], rllv2.at[j], semR).wait()

    def selb(k, _):
        row = k // VPR
        off = (k % VPR) * 16
        mvec = mval[row, pl.ds(off, 16)]
        a = llv2[row, pl.ds(off, 16)]
        b = rllv2[row, pl.ds(off, 16)]
        llv2[row, pl.ds(off, 16)] = jnp.where(mvec > 0, b, a)
        return 0
    lax.fori_loop(0, CB // 16, selb, 0)
    pltpu.sync_copy(llv2, llo.at[pl.ds(wid * NK, NK)])

    def tbody(t, _):
        for (src_tab, res_tab, out_tab) in ((fm, rm, om), (fv, rv, ov)):
            for j in range(NK):
                pltpu.async_copy(src_tab.at[t].at[r2d.at[j]],
                                 bufFR.at[pl.ds(j * K, K)], semF)
                pltpu.async_copy(res_tab.at[t].at[r2d.at[j]],
                                 bufFR.at[pl.ds(CB + j * K, K)], semR)
            for j in range(NK):
                pltpu.make_async_copy(src_tab.at[t].at[r2d.at[j]],
                                      bufFR.at[pl.ds(j * K, K)], semF).wait()
            pltpu.sync_copy(bufFR.at[pl.ds(0, CB)],
                            spmem_out.at[pl.ds(slot, CB)])
            for j in range(NK):
                pltpu.make_async_copy(res_tab.at[t].at[r2d.at[j]],
                                      bufFR.at[pl.ds(CB + j * K, K)],
                                      semR).wait()
            for j in range(NK):
                pltpu.async_copy(bufFR.at[pl.ds(CB + j * K, K)],
                                 spmem_out.at[dst2.at[j]], semS)
            for j in range(NK):
                pltpu.make_async_copy(bufFR.at[pl.ds(CB + j * K, K)],
                                      spmem_out.at[dst2.at[j]], semS).wait()
            pltpu.sync_copy(spmem_out.at[pl.ds(slot, CB)],
                            out_tab.at[t, pl.ds(base, CB)])
        return 0

    lax.fori_loop(0, T, tbody, 0)


_mesh = plsc.VectorSubcoreMesh(core_axis_name="c", subcore_axis_name="s")

_sc_call = functools.partial(
    pl.kernel,
    out_type=[
        jax.ShapeDtypeStruct((B // K, K), jnp.float32),
        jax.ShapeDtypeStruct((T, B, D), jnp.float32),
        jax.ShapeDtypeStruct((T, B, D), jnp.float32),
    ],
    mesh=_mesh,
    compiler_params=pltpu.CompilerParams(use_tc_tiling_on_sc=False),
    scratch_types=[
        pltpu.VMEM((NK, K), jnp.int32),
        pltpu.VMEM((NK, K), jnp.int32),
        pltpu.VMEM((EROWS, K), jnp.int32),
        pltpu.VMEM((K,), jnp.int32),
        pltpu.VMEM((B // NS,), jnp.int32),
        pltpu.VMEM((NK, K), jnp.int32),
        pltpu.VMEM((2 * CB, D), jnp.float32),
        pltpu.VMEM((NK, K), jnp.float32),
        pltpu.VMEM((NK, K), jnp.float32),
        pltpu.VMEM_SHARED((B,), jnp.int32),
        pltpu.VMEM_SHARED((NS * (CB + K), D), jnp.float32),
        pltpu.SemaphoreType.DMA,
        pltpu.SemaphoreType.DMA,
        pltpu.SemaphoreType.DMA,
    ],
)(_body)


def kernel(loglikelihood, filter_means, filter_vars, res_loglikelihood,
           res_means, res_vars, exchange_indices, resample_indices):
    eix = exchange_indices.astype(jnp.int32).reshape(B_EX // K, K)
    rix = resample_indices.astype(jnp.int32).reshape(B // K, K)
    llo, om, ov = _sc_call(loglikelihood, filter_means, filter_vars,
                           res_loglikelihood, res_means, res_vars, eix, rix)
    return (llo.reshape(B), om, ov)


# speedup vs baseline: 1.1302x; 1.0721x over previous
import functools

import jax
import jax.numpy as jnp
from jax import lax
from jax.experimental import pallas as pl
from jax.experimental.pallas import tpu as pltpu
from jax.experimental.pallas import tpu_sc as plsc

T, B, D = 20, 65536, 16
B_EX = 16384
NC, NS = 2, 16
NW = NC * NS
CB = B // NW
K = 128
NK = CB // K
EROWS = B_EX // NS // K
VPR = K // 16


def _body_a(ll, fm, rll, rm, eix, rix, llo, om,
            r2d, mval, eidx2, ones_v, zeros_v, dst2, bufFR,
            llv2, rllv2, mask_spm, spmem_out, semF, semR, semS):
    cid = lax.axis_index("c")
    sid = lax.axis_index("s")
    wid = sid * NC + cid
    base = wid * CB
    iota16 = lax.broadcasted_iota(jnp.int32, (16,), 0)

    def fill(ref, nvec, val):
        def f(i, _):
            ref[pl.ds(i * 16, 16)] = jnp.full((16,), val, jnp.int32)
            return 0
        lax.fori_loop(0, nvec, f, 0)

    fill(ones_v, K // 16, 1)
    fill(zeros_v, (B // NS) // 16, 0)

    pltpu.sync_copy(rix.at[pl.ds(wid * NK, NK)], r2d)

    pltpu.sync_copy(zeros_v, mask_spm.at[pl.ds(sid * (B // NS), B // NS)])
    plsc.subcore_barrier()
    pltpu.sync_copy(eix.at[pl.ds(sid * EROWS, EROWS)], eidx2)
    for j in range(EROWS):
        pltpu.sync_copy(ones_v, mask_spm.at[eidx2.at[j]], add=True)
    plsc.subcore_barrier()

    for j in range(NK):
        pltpu.async_copy(mask_spm.at[r2d.at[j]], mval.at[j], semF)
    for j in range(NK):
        pltpu.make_async_copy(mask_spm.at[r2d.at[j]], mval.at[j], semF).wait()

    slot = sid * (CB + K)
    def cbody(k, _):
        row = k // VPR
        off = (k % VPR) * 16
        mvec = mval[row, pl.ds(off, 16)]
        jloc = k * 16 + iota16
        dump = slot + CB + jnp.bitwise_and(jloc, K - 1)
        dst2[row, pl.ds(off, 16)] = jnp.where(mvec > 0, slot + jloc, dump)
        return 0
    lax.fori_loop(0, CB // 16, cbody, 0)

    for j in range(NK):
        pltpu.async_copy(ll.at[r2d.at[j]], llv2.at[j], semF)
        pltpu.async_copy(rll.at[r2d.at[j]], rllv2.at[j], semR)
    for j in range(NK):
        pltpu.make_async_copy(ll.at[r2d.at[j]], llv2.at[j], semF).wait()
        pltpu.make_async_copy(rll.at[r2d.at[j]], rllv2.at[j], semR).wait()

    def selb(k, _):
        row = k // VPR
        off = (k % VPR) * 16
        mvec = mval[row, pl.ds(off, 16)]
        a = llv2[row, pl.ds(off, 16)]
        b = rllv2[row, pl.ds(off, 16)]
        llv2[row, pl.ds(off, 16)] = jnp.where(mvec > 0, b, a)
        return 0
    lax.fori_loop(0, CB // 16, selb, 0)
    pltpu.sync_copy(llv2, llo.at[pl.ds(wid * NK, NK)])

    def tbody(t, _):
        for (src_tab, res_tab, out_tab) in ((fm, rm, om),):
            for j in range(NK):
                pltpu.async_copy(src_tab.at[t].at[r2d.at[j]],
                                 bufFR.at[pl.ds(j * K, K)], semF)
                pltpu.async_copy(res_tab.at[t].at[r2d.at[j]],
                                 bufFR.at[pl.ds(CB + j * K, K)], semR)
            for j in range(NK):
                pltpu.make_async_copy(src_tab.at[t].at[r2d.at[j]],
                                      bufFR.at[pl.ds(j * K, K)], semF).wait()
            pltpu.sync_copy(bufFR.at[pl.ds(0, CB)],
                            spmem_out.at[pl.ds(slot, CB)])
            for j in range(NK):
                pltpu.make_async_copy(res_tab.at[t].at[r2d.at[j]],
                                      bufFR.at[pl.ds(CB + j * K, K)],
                                      semR).wait()
            for j in range(NK):
                pltpu.async_copy(bufFR.at[pl.ds(CB + j * K, K)],
                                 spmem_out.at[dst2.at[j]], semS)
            for j in range(NK):
                pltpu.make_async_copy(bufFR.at[pl.ds(CB + j * K, K)],
                                      spmem_out.at[dst2.at[j]], semS).wait()
            pltpu.sync_copy(spmem_out.at[pl.ds(slot, CB)],
                            out_tab.at[t, pl.ds(base, CB)])
        return 0

    lax.fori_loop(0, T, tbody, 0)


def _body_b(fv, rv, eix, rix, ov,
            r2d, mval, eidx2, ones_v, zeros_v, dst2, bufFR,
            mask_spm, spmem_out, semF, semR, semS):
    cid = lax.axis_index("c")
    sid = lax.axis_index("s")
    wid = sid * NC + cid
    base = wid * CB
    iota16 = lax.broadcasted_iota(jnp.int32, (16,), 0)

    def fill(ref, nvec, val):
        def f(i, _):
            ref[pl.ds(i * 16, 16)] = jnp.full((16,), val, jnp.int32)
            return 0
        lax.fori_loop(0, nvec, f, 0)

    fill(ones_v, K // 16, 1)
    fill(zeros_v, (B // NS) // 16, 0)

    pltpu.sync_copy(rix.at[pl.ds(wid * NK, NK)], r2d)

    pltpu.sync_copy(zeros_v, mask_spm.at[pl.ds(sid * (B // NS), B // NS)])
    plsc.subcore_barrier()
    pltpu.sync_copy(eix.at[pl.ds(sid * EROWS, EROWS)], eidx2)
    for j in range(EROWS):
        pltpu.sync_copy(ones_v, mask_spm.at[eidx2.at[j]], add=True)
    plsc.subcore_barrier()

    for j in range(NK):
        pltpu.async_copy(mask_spm.at[r2d.at[j]], mval.at[j], semF)
    for j in range(NK):
        pltpu.make_async_copy(mask_spm.at[r2d.at[j]], mval.at[j], semF).wait()

    slot = sid * (CB + K)
    def cbody(k, _):
        row = k // VPR
        off = (k % VPR) * 16
        mvec = mval[row, pl.ds(off, 16)]
        jloc = k * 16 + iota16
        dump = slot + CB + jnp.bitwise_and(jloc, K - 1)
        dst2[row, pl.ds(off, 16)] = jnp.where(mvec > 0, slot + jloc, dump)
        return 0
    lax.fori_loop(0, CB // 16, cbody, 0)

    def tbody(t, _):
        for (src_tab, res_tab, out_tab) in ((fv, rv, ov),):
            for j in range(NK):
                pltpu.async_copy(src_tab.at[t].at[r2d.at[j]],
                                 bufFR.at[pl.ds(j * K, K)], semF)
                pltpu.async_copy(res_tab.at[t].at[r2d.at[j]],
                                 bufFR.at[pl.ds(CB + j * K, K)], semR)
            for j in range(NK):
                pltpu.make_async_copy(src_tab.at[t].at[r2d.at[j]],
                                      bufFR.at[pl.ds(j * K, K)], semF).wait()
            pltpu.sync_copy(bufFR.at[pl.ds(0, CB)],
                            spmem_out.at[pl.ds(slot, CB)])
            for j in range(NK):
                pltpu.make_async_copy(res_tab.at[t].at[r2d.at[j]],
                                      bufFR.at[pl.ds(CB + j * K, K)],
                                      semR).wait()
            for j in range(NK):
                pltpu.async_copy(bufFR.at[pl.ds(CB + j * K, K)],
                                 spmem_out.at[dst2.at[j]], semS)
            for j in range(NK):
                pltpu.make_async_copy(bufFR.at[pl.ds(CB + j * K, K)],
                                      spmem_out.at[dst2.at[j]], semS).wait()
            pltpu.sync_copy(spmem_out.at[pl.ds(slot, CB)],
                            out_tab.at[t, pl.ds(base, CB)])
        return 0

    lax.fori_loop(0, T, tbody, 0)


_mesh = plsc.VectorSubcoreMesh(core_axis_name="c", subcore_axis_name="s")

_scratch_a = [
    pltpu.VMEM((NK, K), jnp.int32),
    pltpu.VMEM((NK, K), jnp.int32),
    pltpu.VMEM((EROWS, K), jnp.int32),
    pltpu.VMEM((K,), jnp.int32),
    pltpu.VMEM((B // NS,), jnp.int32),
    pltpu.VMEM((NK, K), jnp.int32),
    pltpu.VMEM((2 * CB, D), jnp.float32),
    pltpu.VMEM((NK, K), jnp.float32),
    pltpu.VMEM((NK, K), jnp.float32),
    pltpu.VMEM_SHARED((B,), jnp.int32),
    pltpu.VMEM_SHARED((NS * (CB + K), D), jnp.float32),
    pltpu.SemaphoreType.DMA,
    pltpu.SemaphoreType.DMA,
    pltpu.SemaphoreType.DMA,
]
_scratch_b = [
    pltpu.VMEM((NK, K), jnp.int32),
    pltpu.VMEM((NK, K), jnp.int32),
    pltpu.VMEM((EROWS, K), jnp.int32),
    pltpu.VMEM((K,), jnp.int32),
    pltpu.VMEM((B // NS,), jnp.int32),
    pltpu.VMEM((NK, K), jnp.int32),
    pltpu.VMEM((2 * CB, D), jnp.float32),
    pltpu.VMEM_SHARED((B,), jnp.int32),
    pltpu.VMEM_SHARED((NS * (CB + K), D), jnp.float32),
    pltpu.SemaphoreType.DMA,
    pltpu.SemaphoreType.DMA,
    pltpu.SemaphoreType.DMA,
]

_call_a = functools.partial(
    pl.kernel,
    out_type=[
        jax.ShapeDtypeStruct((B // K, K), jnp.float32),
        jax.ShapeDtypeStruct((T, B, D), jnp.float32),
    ],
    mesh=_mesh,
    compiler_params=pltpu.CompilerParams(use_tc_tiling_on_sc=False),
    scratch_types=_scratch_a,
)(_body_a)

_call_b = functools.partial(
    pl.kernel,
    out_type=[
        jax.ShapeDtypeStruct((T, B, D), jnp.float32),
    ],
    mesh=_mesh,
    compiler_params=pltpu.CompilerParams(use_tc_tiling_on_sc=False),
    scratch_types=_scratch_b,
)(_body_b)


def kernel(loglikelihood, filter_means, filter_vars, res_loglikelihood,
           res_means, res_vars, exchange_indices, resample_indices):
    eix = exchange_indices.astype(jnp.int32).reshape(B_EX // K, K)
    rix = resample_indices.astype(jnp.int32).reshape(B // K, K)
    llo, om = _call_a(loglikelihood, filter_means, res_loglikelihood,
                      res_means, eix, rix)
    (ov,) = _call_b(filter_vars, res_vars, eix, rix)
    return (llo.reshape(B), om, ov)
